# Initial kernel scaffold; baseline (speedup 1.0000x reference)
import jax
import jax.numpy as jnp
from jax.experimental import pallas as pl


def kernel(xyz1, xyz2, features1, features2, W1, b1, g1, be1, W2, b2, g2, be2):
    raise NotImplementedError("write your pallas kernel here")


# speedup vs baseline: 22.9347x; 22.9347x over previous
import jax
import jax.numpy as jnp
from jax.experimental import pallas as pl


_BLK = 512


def _stage1_body(x1_ref, x2_ref, f1_ref, f2_ref, w1a_ref, w1b_ref,
                 y1_ref, st_ref):
    i = pl.program_id(0)
    blk = x1_ref.shape[2]
    s = x2_ref.shape[2]

    d = jnp.zeros((blk, s), jnp.float32)
    for c in range(3):
        a = x1_ref[0, c, :]
        b = x2_ref[0, c, :]
        diff = a[:, None] - b[None, :]
        d = d + diff * diff

    ids = jax.lax.broadcasted_iota(jnp.int32, (blk, s), 1)
    big = jnp.float32(3.4e38)

    recips = []
    onehots = []
    for _ in range(3):
        vk = jnp.min(d, axis=1, keepdims=True)
        first = jnp.min(jnp.where(d == vk, ids, s), axis=1, keepdims=True)
        onek = ids == first
        recips.append(1.0 / (vk + 1e-8))
        onehots.append(onek)
        d = jnp.where(onek, big, d)

    norm = recips[0] + recips[1] + recips[2]
    amat = jnp.where(onehots[0], recips[0] / norm, 0.0)
    amat = amat + jnp.where(onehots[1], recips[1] / norm, 0.0)
    amat = amat + jnp.where(onehots[2], recips[2] / norm, 0.0)

    interp = jnp.dot(amat, f2_ref[0], preferred_element_type=jnp.float32)
    y1 = (jnp.dot(interp, w1a_ref[...], preferred_element_type=jnp.float32)
          + jnp.dot(f1_ref[...], w1b_ref[...],
                    preferred_element_type=jnp.float32))
    y1_ref[...] = y1

    @pl.when(i == 0)
    def _():
        st_ref[...] = jnp.zeros_like(st_ref)

    ssum = jnp.sum(y1, axis=0)
    ssq = jnp.sum(y1 * y1, axis=0)
    st_ref[...] += jnp.concatenate([ssum[None, :], ssq[None, :]], axis=0)


def _stage2_body(y1_ref, sc_ref, sh_ref, w2_ref, y2_ref, st_ref):
    i = pl.program_id(0)
    h = jnp.maximum(y1_ref[...] * sc_ref[...] + sh_ref[...], 0.0)
    y2 = jnp.dot(h, w2_ref[...], preferred_element_type=jnp.float32)
    y2_ref[...] = y2

    @pl.when(i == 0)
    def _():
        st_ref[...] = jnp.zeros_like(st_ref)

    ssum = jnp.sum(y2, axis=0)
    ssq = jnp.sum(y2 * y2, axis=0)
    st_ref[...] += jnp.concatenate([ssum[None, :], ssq[None, :]], axis=0)


def _stage3_body(y2_ref, sc_ref, sh_ref, out_ref):
    out_ref[...] = jnp.maximum(y2_ref[...] * sc_ref[...] + sh_ref[...], 0.0)


def kernel(xyz1, xyz2, features1, features2, W1, b1, g1, be1, W2, b2, g2, be2):
    B, N, _ = xyz1.shape
    S = xyz2.shape[1]
    C1 = features1.shape[2]
    C2 = features2.shape[2]
    H1 = W1.shape[0]
    H2 = W2.shape[0]
    M = B * N
    blk = _BLK
    nblk_b = N // blk
    grid = M // blk

    x1t = jnp.transpose(xyz1, (0, 2, 1))
    x2t = jnp.transpose(xyz2, (0, 2, 1))
    f1r = features1.reshape(M, C1)
    w1a_t = jnp.transpose(W1[:, :C2])
    w1b_t = jnp.transpose(W1[:, C2:])
    w2_t = jnp.transpose(W2)

    y1, st1 = pl.pallas_call(
        _stage1_body,
        grid=(grid,),
        in_specs=[
            pl.BlockSpec((1, 3, blk), lambda i: (i // nblk_b, 0, i % nblk_b)),
            pl.BlockSpec((1, 3, S), lambda i: (i // nblk_b, 0, 0)),
            pl.BlockSpec((blk, C1), lambda i: (i, 0)),
            pl.BlockSpec((1, S, C2), lambda i: (i // nblk_b, 0, 0)),
            pl.BlockSpec((C2, H1), lambda i: (0, 0)),
            pl.BlockSpec((C1, H1), lambda i: (0, 0)),
        ],
        out_specs=[
            pl.BlockSpec((blk, H1), lambda i: (i, 0)),
            pl.BlockSpec((2, H1), lambda i: (0, 0)),
        ],
        out_shape=[
            jax.ShapeDtypeStruct((M, H1), jnp.float32),
            jax.ShapeDtypeStruct((2, H1), jnp.float32),
        ],
    )(x1t, x2t, f1r, features2, w1a_t, w1b_t)

    eps = 1e-5
    mean1 = st1[0] / M
    var1 = st1[1] / M - mean1 * mean1
    scale1 = (g1 / jnp.sqrt(var1 + eps))[None, :]
    shift1 = (be1 - mean1 * (g1 / jnp.sqrt(var1 + eps)))[None, :]

    y2, st2 = pl.pallas_call(
        _stage2_body,
        grid=(grid,),
        in_specs=[
            pl.BlockSpec((blk, H1), lambda i: (i, 0)),
            pl.BlockSpec((1, H1), lambda i: (0, 0)),
            pl.BlockSpec((1, H1), lambda i: (0, 0)),
            pl.BlockSpec((H1, H2), lambda i: (0, 0)),
        ],
        out_specs=[
            pl.BlockSpec((blk, H2), lambda i: (i, 0)),
            pl.BlockSpec((2, H2), lambda i: (0, 0)),
        ],
        out_shape=[
            jax.ShapeDtypeStruct((M, H2), jnp.float32),
            jax.ShapeDtypeStruct((2, H2), jnp.float32),
        ],
    )(y1, scale1, shift1, w2_t)

    mean2 = st2[0] / M
    var2 = st2[1] / M - mean2 * mean2
    scale2 = (g2 / jnp.sqrt(var2 + eps))[None, :]
    shift2 = (be2 - mean2 * (g2 / jnp.sqrt(var2 + eps)))[None, :]

    out = pl.pallas_call(
        _stage3_body,
        grid=(grid,),
        in_specs=[
            pl.BlockSpec((blk, H2), lambda i: (i, 0)),
            pl.BlockSpec((1, H2), lambda i: (0, 0)),
            pl.BlockSpec((1, H2), lambda i: (0, 0)),
        ],
        out_specs=pl.BlockSpec((blk, H2), lambda i: (i, 0)),
        out_shape=jax.ShapeDtypeStruct((M, H2), jnp.float32),
    )(y2, scale2, shift2)

    return out.reshape(B, N, H2)


# speedup vs baseline: 27.8146x; 1.2128x over previous
import jax
import jax.numpy as jnp
from jax.experimental import pallas as pl


_BLK = 512


def _stage1_body(x1_ref, x2_ref, f1_ref, f2_ref, w1a_ref, w1b_ref,
                 y1_ref, st_ref):
    i = pl.program_id(0)
    blk = x1_ref.shape[1]
    s = x2_ref.shape[2]

    d = jnp.zeros((blk, s), jnp.float32)
    for c in range(3):
        a = x1_ref[0, :, c:c + 1]
        b = x2_ref[0, c, :]
        diff = a - b[None, :]
        d = d + diff * diff

    big = jnp.float32(3.4e38)
    recips = []
    onehots = []
    for _ in range(3):
        vk = jnp.min(d, axis=1, keepdims=True)
        onek = d == vk
        recips.append(1.0 / (vk + 1e-8))
        onehots.append(onek)
        d = jnp.where(onek, big, d)

    norm = recips[0] + recips[1] + recips[2]
    amat = jnp.where(onehots[0], recips[0] / norm, 0.0)
    amat = amat + jnp.where(onehots[1], recips[1] / norm, 0.0)
    amat = amat + jnp.where(onehots[2], recips[2] / norm, 0.0)

    interp = jnp.dot(amat, f2_ref[0], preferred_element_type=jnp.float32)
    y1 = (jnp.dot(interp, w1a_ref[...], preferred_element_type=jnp.float32)
          + jnp.dot(f1_ref[...], w1b_ref[...],
                    preferred_element_type=jnp.float32))
    y1_ref[...] = y1

    @pl.when(i == 0)
    def _():
        st_ref[...] = jnp.zeros_like(st_ref)

    ssum = jnp.sum(y1, axis=0)
    ssq = jnp.sum(y1 * y1, axis=0)
    st_ref[...] += jnp.concatenate([ssum[None, :], ssq[None, :]], axis=0)


def _bn_coeffs(st_ref, g_ref, be_ref, m):
    mean = st_ref[0:1, :] * (1.0 / m)
    var = st_ref[1:2, :] * (1.0 / m) - mean * mean
    scale = g_ref[...] * jax.lax.rsqrt(var + 1e-5)
    shift = be_ref[...] - mean * scale
    return scale, shift


def _stage2_body(m, y1_ref, st1_ref, g1_ref, be1_ref, w2_ref,
                 y2_ref, st_ref):
    i = pl.program_id(0)
    scale, shift = _bn_coeffs(st1_ref, g1_ref, be1_ref, m)
    h = jnp.maximum(y1_ref[...] * scale + shift, 0.0)
    y2 = jnp.dot(h, w2_ref[...], preferred_element_type=jnp.float32)
    y2_ref[...] = y2

    @pl.when(i == 0)
    def _():
        st_ref[...] = jnp.zeros_like(st_ref)

    ssum = jnp.sum(y2, axis=0)
    ssq = jnp.sum(y2 * y2, axis=0)
    st_ref[...] += jnp.concatenate([ssum[None, :], ssq[None, :]], axis=0)


def _stage3_body(m, y2_ref, st2_ref, g2_ref, be2_ref, out_ref):
    scale, shift = _bn_coeffs(st2_ref, g2_ref, be2_ref, m)
    out_ref[...] = jnp.maximum(y2_ref[...] * scale + shift, 0.0)


def kernel(xyz1, xyz2, features1, features2, W1, b1, g1, be1, W2, b2, g2, be2):
    import functools

    B, N, _ = xyz1.shape
    S = xyz2.shape[1]
    C1 = features1.shape[2]
    C2 = features2.shape[2]
    H1 = W1.shape[0]
    H2 = W2.shape[0]
    M = B * N
    blk = _BLK
    nblk_b = N // blk
    grid = M // blk

    x2t = jnp.transpose(xyz2, (0, 2, 1))
    f1r = features1.reshape(M, C1)
    w1a_t = jnp.transpose(W1[:, :C2])
    w1b_t = jnp.transpose(W1[:, C2:])
    w2_t = jnp.transpose(W2)

    y1, st1 = pl.pallas_call(
        _stage1_body,
        grid=(grid,),
        in_specs=[
            pl.BlockSpec((1, blk, 3), lambda i: (i // nblk_b, i % nblk_b, 0)),
            pl.BlockSpec((1, 3, S), lambda i: (i // nblk_b, 0, 0)),
            pl.BlockSpec((blk, C1), lambda i: (i, 0)),
            pl.BlockSpec((1, S, C2), lambda i: (i // nblk_b, 0, 0)),
            pl.BlockSpec((C2, H1), lambda i: (0, 0)),
            pl.BlockSpec((C1, H1), lambda i: (0, 0)),
        ],
        out_specs=[
            pl.BlockSpec((blk, H1), lambda i: (i, 0)),
            pl.BlockSpec((2, H1), lambda i: (0, 0)),
        ],
        out_shape=[
            jax.ShapeDtypeStruct((M, H1), jnp.float32),
            jax.ShapeDtypeStruct((2, H1), jnp.float32),
        ],
    )(xyz1, x2t, f1r, features2, w1a_t, w1b_t)

    y2, st2 = pl.pallas_call(
        functools.partial(_stage2_body, float(M)),
        grid=(grid,),
        in_specs=[
            pl.BlockSpec((blk, H1), lambda i: (i, 0)),
            pl.BlockSpec((2, H1), lambda i: (0, 0)),
            pl.BlockSpec((1, H1), lambda i: (0, 0)),
            pl.BlockSpec((1, H1), lambda i: (0, 0)),
            pl.BlockSpec((H1, H2), lambda i: (0, 0)),
        ],
        out_specs=[
            pl.BlockSpec((blk, H2), lambda i: (i, 0)),
            pl.BlockSpec((2, H2), lambda i: (0, 0)),
        ],
        out_shape=[
            jax.ShapeDtypeStruct((M, H2), jnp.float32),
            jax.ShapeDtypeStruct((2, H2), jnp.float32),
        ],
    )(y1, st1, g1.reshape(1, H1), be1.reshape(1, H1), w2_t)

    out = pl.pallas_call(
        functools.partial(_stage3_body, float(M)),
        grid=(grid,),
        in_specs=[
            pl.BlockSpec((blk, H2), lambda i: (i, 0)),
            pl.BlockSpec((2, H2), lambda i: (0, 0)),
            pl.BlockSpec((1, H2), lambda i: (0, 0)),
            pl.BlockSpec((1, H2), lambda i: (0, 0)),
        ],
        out_specs=pl.BlockSpec((blk, H2), lambda i: (i, 0)),
        out_shape=jax.ShapeDtypeStruct((M, H2), jnp.float32),
    )(y2, st2, g2.reshape(1, H2), be2.reshape(1, H2))

    return out.reshape(B, N, H2)


# speedup vs baseline: 29.4948x; 1.0604x over previous
import jax
import jax.numpy as jnp
from jax.experimental import pallas as pl


_BLK = 512


def _stage1_body(x1_ref, x2_ref, f1_ref, f2_ref, w1a_ref, w1b_ref,
                 y1_ref, st_ref):
    i = pl.program_id(0)
    blk = x1_ref.shape[2]
    s = x2_ref.shape[2]

    d = jnp.zeros((blk, s), jnp.float32)
    for c in range(3):
        a = x1_ref[0, c, :]
        b = x2_ref[0, c, :]
        diff = a[:, None] - b[None, :]
        d = d + diff * diff

    big = jnp.float32(3.4e38)
    recips = []
    onehots = []
    for _ in range(3):
        vk = jnp.min(d, axis=1, keepdims=True)
        onek = d == vk
        recips.append(1.0 / (vk + 1e-8))
        onehots.append(onek)
        d = jnp.where(onek, big, d)

    norm = recips[0] + recips[1] + recips[2]
    amat = jnp.where(onehots[0], recips[0] / norm, 0.0)
    amat = amat + jnp.where(onehots[1], recips[1] / norm, 0.0)
    amat = amat + jnp.where(onehots[2], recips[2] / norm, 0.0)

    interp = jnp.dot(amat, f2_ref[0], preferred_element_type=jnp.float32)
    y1 = (jnp.dot(interp, w1a_ref[...], preferred_element_type=jnp.float32)
          + jnp.dot(f1_ref[...], w1b_ref[...],
                    preferred_element_type=jnp.float32))
    y1_ref[...] = y1

    @pl.when(i == 0)
    def _():
        st_ref[...] = jnp.zeros_like(st_ref)

    ssum = jnp.sum(y1, axis=0)
    ssq = jnp.sum(y1 * y1, axis=0)
    st_ref[...] += jnp.concatenate([ssum[None, :], ssq[None, :]], axis=0)


def _bn_coeffs(st_ref, g_ref, be_ref, m):
    mean = st_ref[0:1, :] * (1.0 / m)
    var = st_ref[1:2, :] * (1.0 / m) - mean * mean
    scale = g_ref[...] * jax.lax.rsqrt(var + 1e-5)
    shift = be_ref[...] - mean * scale
    return scale, shift


def _stage2_body(m, y1_ref, st1_ref, g1_ref, be1_ref, w2_ref,
                 y2_ref, st_ref):
    i = pl.program_id(0)
    scale, shift = _bn_coeffs(st1_ref, g1_ref, be1_ref, m)
    h = jnp.maximum(y1_ref[...] * scale + shift, 0.0)
    y2 = jnp.dot(h, w2_ref[...], preferred_element_type=jnp.float32)
    y2_ref[...] = y2

    @pl.when(i == 0)
    def _():
        st_ref[...] = jnp.zeros_like(st_ref)

    ssum = jnp.sum(y2, axis=0)
    ssq = jnp.sum(y2 * y2, axis=0)
    st_ref[...] += jnp.concatenate([ssum[None, :], ssq[None, :]], axis=0)


def _stage3_body(m, y2_ref, st2_ref, g2_ref, be2_ref, out_ref):
    scale, shift = _bn_coeffs(st2_ref, g2_ref, be2_ref, m)
    out_ref[...] = jnp.maximum(y2_ref[...] * scale + shift, 0.0)


def kernel(xyz1, xyz2, features1, features2, W1, b1, g1, be1, W2, b2, g2, be2):
    import functools

    B, N, _ = xyz1.shape
    S = xyz2.shape[1]
    C1 = features1.shape[2]
    C2 = features2.shape[2]
    H1 = W1.shape[0]
    H2 = W2.shape[0]
    M = B * N
    blk = _BLK
    nblk_b = N // blk
    grid = M // blk

    x1t = jnp.transpose(xyz1, (0, 2, 1))
    x2t = jnp.transpose(xyz2, (0, 2, 1))
    f1r = features1.reshape(M, C1)
    w1a_t = jnp.transpose(W1[:, :C2])
    w1b_t = jnp.transpose(W1[:, C2:])
    w2_t = jnp.transpose(W2)

    y1, st1 = pl.pallas_call(
        _stage1_body,
        grid=(grid,),
        in_specs=[
            pl.BlockSpec((1, 3, blk), lambda i: (i // nblk_b, 0, i % nblk_b)),
            pl.BlockSpec((1, 3, S), lambda i: (i // nblk_b, 0, 0)),
            pl.BlockSpec((blk, C1), lambda i: (i, 0)),
            pl.BlockSpec((1, S, C2), lambda i: (i // nblk_b, 0, 0)),
            pl.BlockSpec((C2, H1), lambda i: (0, 0)),
            pl.BlockSpec((C1, H1), lambda i: (0, 0)),
        ],
        out_specs=[
            pl.BlockSpec((blk, H1), lambda i: (i, 0)),
            pl.BlockSpec((2, H1), lambda i: (0, 0)),
        ],
        out_shape=[
            jax.ShapeDtypeStruct((M, H1), jnp.float32),
            jax.ShapeDtypeStruct((2, H1), jnp.float32),
        ],
    )(x1t, x2t, f1r, features2, w1a_t, w1b_t)

    y2, st2 = pl.pallas_call(
        functools.partial(_stage2_body, float(M)),
        grid=(grid,),
        in_specs=[
            pl.BlockSpec((blk, H1), lambda i: (i, 0)),
            pl.BlockSpec((2, H1), lambda i: (0, 0)),
            pl.BlockSpec((1, H1), lambda i: (0, 0)),
            pl.BlockSpec((1, H1), lambda i: (0, 0)),
            pl.BlockSpec((H1, H2), lambda i: (0, 0)),
        ],
        out_specs=[
            pl.BlockSpec((blk, H2), lambda i: (i, 0)),
            pl.BlockSpec((2, H2), lambda i: (0, 0)),
        ],
        out_shape=[
            jax.ShapeDtypeStruct((M, H2), jnp.float32),
            jax.ShapeDtypeStruct((2, H2), jnp.float32),
        ],
    )(y1, st1, g1.reshape(1, H1), be1.reshape(1, H1), w2_t)

    out = pl.pallas_call(
        functools.partial(_stage3_body, float(M)),
        grid=(grid,),
        in_specs=[
            pl.BlockSpec((blk, H2), lambda i: (i, 0)),
            pl.BlockSpec((2, H2), lambda i: (0, 0)),
            pl.BlockSpec((1, H2), lambda i: (0, 0)),
            pl.BlockSpec((1, H2), lambda i: (0, 0)),
        ],
        out_specs=pl.BlockSpec((blk, H2), lambda i: (i, 0)),
        out_shape=jax.ShapeDtypeStruct((M, H2), jnp.float32),
    )(y2, st2, g2.reshape(1, H2), be2.reshape(1, H2))

    return out.reshape(B, N, H2)
